# Initial kernel scaffold; baseline (speedup 1.0000x reference)
import jax
import jax.numpy as jnp
from jax.experimental import pallas as pl


def kernel(nodes, features, edge_index, W, b):
    raise NotImplementedError("write your pallas kernel here")


# speedup vs baseline: 6.3326x; 6.3326x over previous
import functools

import jax
import jax.numpy as jnp
from jax import lax
from jax.experimental import pallas as pl
from jax.experimental.pallas import tpu as pltpu
from jax.experimental.pallas import tpu_sc as plsc

N_NODES = 10000
D = 128
N_EDGES = 320000
BATCH = 4096

NC = 2
NS = 16
NW = NC * NS

EDGES_PER_TILE = N_EDGES // NW
CHUNK = 80
N_CHUNKS = EDGES_PER_TILE // CHUNK
ROWS_PER_TILE = N_NODES // NS
ZCHUNK = 125
DZ = 624
BCHUNK = 128
B_PER_TILE = BATCH // NS


def _sc_aggregate(features, src, dst, nodes):
    mesh = plsc.VectorSubcoreMesh(core_axis_name="c", subcore_axis_name="s")

    @functools.partial(
        pl.kernel,
        mesh=mesh,
        out_type=[
            jax.ShapeDtypeStruct((NC, BATCH, D), jnp.float32),
            jax.ShapeDtypeStruct((NC * BATCH,), jnp.float32),
            jax.ShapeDtypeStruct((BATCH, D), jnp.float32),
        ],
        scratch_types=[
            pltpu.VMEM_SHARED((N_NODES, D), jnp.float32),
            pltpu.VMEM_SHARED((N_NODES,), jnp.float32),
            pltpu.VMEM((CHUNK,), jnp.int32),
            pltpu.VMEM((CHUNK,), jnp.int32),
            pltpu.VMEM((CHUNK, D), jnp.float32),
            pltpu.VMEM((CHUNK,), jnp.float32),
            pltpu.VMEM((ZCHUNK, D), jnp.float32),
            pltpu.VMEM((DZ,), jnp.float32),
            pltpu.VMEM((BCHUNK,), jnp.int32),
            pltpu.VMEM((BCHUNK, D), jnp.float32),
            pltpu.VMEM((BCHUNK,), jnp.float32),
            pltpu.SemaphoreType.DMA,
        ],
    )
    def agg(feat_hbm, src_hbm, dst_hbm, nodes_hbm, part_hbm, pdeg_hbm,
            self_hbm, acc_sh, deg_sh, src_v, dst_v, rows_v, ones_v, zero_v,
            zvec_v, bidx_v, brow_v, bdeg_v, sem):
        cid = lax.axis_index("c")
        sid = lax.axis_index("s")
        wid = sid * NC + cid

        zeros16 = jnp.zeros((16,), jnp.float32)
        ones16 = jnp.ones((16,), jnp.float32)

        def init_zero(i, _):
            r = i // (D // 16)
            c = i % (D // 16)
            zero_v[r, pl.ds(c * 16, 16)] = zeros16
            return 0
        lax.fori_loop(0, ZCHUNK * (D // 16), init_zero, 0)

        def init_zvec(i, _):
            zvec_v[pl.ds(i * 16, 16)] = zeros16
            return 0
        lax.fori_loop(0, DZ // 16, init_zvec, 0)

        def init_ones(i, _):
            ones_v[pl.ds(i * 16, 16)] = ones16
            return 0
        lax.fori_loop(0, CHUNK // 16, init_ones, 0)

        def zero_acc(i, _):
            base = sid * ROWS_PER_TILE + i * ZCHUNK
            pltpu.sync_copy(zero_v, acc_sh.at[pl.ds(base, ZCHUNK)])
            return 0
        lax.fori_loop(0, ROWS_PER_TILE // ZCHUNK, zero_acc, 0)

        pltpu.sync_copy(zvec_v, deg_sh.at[pl.ds(sid * DZ, DZ)])

        @pl.when(sid == 0)
        def _():
            pltpu.sync_copy(zvec_v.at[pl.ds(0, N_NODES - NS * DZ)],
                            deg_sh.at[pl.ds(NS * DZ, N_NODES - NS * DZ)])

        plsc.subcore_barrier()

        def edge_chunk(j, _):
            base = wid * EDGES_PER_TILE + j * CHUNK
            pltpu.sync_copy(src_hbm.at[pl.ds(base, CHUNK)], src_v)
            pltpu.sync_copy(dst_hbm.at[pl.ds(base, CHUNK)], dst_v)
            pltpu.async_copy(feat_hbm.at[dst_v], rows_v, sem).wait()
            pltpu.sync_copy(rows_v, acc_sh.at[src_v], add=True)
            pltpu.sync_copy(ones_v, deg_sh.at[src_v], add=True)
            return 0
        lax.fori_loop(0, N_CHUNKS, edge_chunk, 0)

        plsc.subcore_barrier()

        def batch_chunk(t, _):
            nbase = sid * B_PER_TILE + t * BCHUNK
            pltpu.sync_copy(nodes_hbm.at[pl.ds(nbase, BCHUNK)], bidx_v)
            pltpu.async_copy(acc_sh.at[bidx_v], brow_v, sem).wait()
            pltpu.sync_copy(brow_v, part_hbm.at[cid, pl.ds(nbase, BCHUNK)])
            pltpu.async_copy(deg_sh.at[bidx_v], bdeg_v, sem).wait()
            pltpu.sync_copy(bdeg_v,
                            pdeg_hbm.at[pl.ds(cid * BATCH + nbase, BCHUNK)])
            return 0
        lax.fori_loop(0, B_PER_TILE // BCHUNK, batch_chunk, 0)

        sbase = wid * (BATCH // NW)
        pltpu.sync_copy(nodes_hbm.at[pl.ds(sbase, BCHUNK)], bidx_v)
        pltpu.async_copy(feat_hbm.at[bidx_v], brow_v, sem).wait()
        pltpu.sync_copy(brow_v, self_hbm.at[pl.ds(sbase, BCHUNK)])

    return agg(features, src, dst, nodes)


def _tc_finish_body(self_ref, part_ref, d0_ref, d1_ref, w1_ref, w2_ref,
                    b_ref, out_ref):
    psum = part_ref[0] + part_ref[1]
    deg = jnp.maximum(d0_ref[...] + d1_ref[...], 1.0)
    mean = psum / deg
    acc = jnp.dot(self_ref[...], w1_ref[...],
                  preferred_element_type=jnp.float32)
    acc += jnp.dot(mean, w2_ref[...], preferred_element_type=jnp.float32)
    out_ref[...] = jnp.maximum(acc + b_ref[...], 0.0)


def _tc_finish(self_feats, partial, d0, d1, W1, W2, b2d):
    blk = 512
    grid = (BATCH // blk,)
    return pl.pallas_call(
        _tc_finish_body,
        grid=grid,
        in_specs=[
            pl.BlockSpec((blk, D), lambda i: (i, 0)),
            pl.BlockSpec((NC, blk, D), lambda i: (0, i, 0)),
            pl.BlockSpec((blk, 1), lambda i: (i, 0)),
            pl.BlockSpec((blk, 1), lambda i: (i, 0)),
            pl.BlockSpec((D, D), lambda i: (0, 0)),
            pl.BlockSpec((D, D), lambda i: (0, 0)),
            pl.BlockSpec((1, D), lambda i: (0, 0)),
        ],
        out_specs=pl.BlockSpec((blk, D), lambda i: (i, 0)),
        out_shape=jax.ShapeDtypeStruct((BATCH, D), jnp.float32),
    )(self_feats, partial, d0, d1, W1, W2, b2d)


def kernel(nodes, features, edge_index, W, b):
    src = edge_index[0]
    dst = edge_index[1]
    partial, pdeg, self_feats = _sc_aggregate(features, src, dst, nodes)
    d0 = pdeg[:BATCH].reshape(BATCH, 1)
    d1 = pdeg[BATCH:].reshape(BATCH, 1)
    W1 = W[:D]
    W2 = W[D:]
    b2d = b.reshape(1, D)
    return _tc_finish(self_feats, partial, d0, d1, W1, W2, b2d)
